# baseline (speedup 1.0000x reference)
import jax
import jax.numpy as jnp
from jax.experimental import pallas as pl
from jax.experimental.pallas import tpu as pltpu


def _lenet_body(x_ref, t1_ref, b1_ref, t2_ref, b2_ref,
                f1w_ref, f1b_ref, f2w_ref, f2b_ref, o_ref, scr_ref):
    bt = x_ref.shape[2]
    i = pl.program_id(0)
    cur = (i % 2) * 12 * bt
    prev = ((i + 1) % 2) * 12 * bt

    def a2(u, owg):
        pieces = [scr_ref[pl.ds(prev + ((u + kh) % 2) * 6 * bt
                                + ((u + kh) // 2) * bt, 4 * bt),
                          128 * owg:128 * owg + 256]
                  for kh in range(5)]
        return jnp.concatenate(pieces, axis=-1)

    t2 = t2_ref[...]
    h2 = [[jnp.dot(a2(u, owg), t2, preferred_element_type=jnp.float32)
           for owg in range(2)] for u in range(2)]
    q2 = []
    for owg in range(2):
        m2 = jnp.maximum(h2[0][owg], h2[1][owg])
        q2.append(jnp.maximum(m2[:, :128], m2[:, 128:]))
    q2 = jnp.concatenate(q2, axis=-1)
    q2 = jnp.maximum(q2 + b2_ref[...], 0.0).astype(jnp.bfloat16)

    hid = sum(jnp.dot(q2[h * bt:(h + 1) * bt, :],
                      f1w_ref[h * 256:(h + 1) * 256, :],
                      preferred_element_type=jnp.float32)
              for h in range(4))
    hid = jnp.maximum(hid + f1b_ref[...], 0.0).astype(jnp.bfloat16)
    out = jnp.dot(hid, f2w_ref[...], preferred_element_type=jnp.float32)
    o_ref[...] = (out + f2b_ref[...]).reshape(bt, 1, 128)

    def a1(s):
        pieces = [x_ref[(s + kh) % 4, (s + kh) // 4:(s + kh) // 4 + 6]
                  .reshape(6 * bt, 28) for kh in range(5)]
        return jnp.concatenate(pieces, axis=-1)

    t1 = t1_ref[...]
    h1 = [jnp.dot(a1(s), t1, preferred_element_type=jnp.float32)
          for s in range(4)]

    def pool1(ha, hb):
        m = jnp.maximum(ha, hb)
        m = jnp.maximum(m[:, :384], m[:, 384:])
        return jnp.maximum(m + b1_ref[...], 0.0).astype(jnp.bfloat16)

    scr_ref[pl.ds(cur, 6 * bt), :] = pool1(h1[0], h1[1])
    scr_ref[pl.ds(cur + 6 * bt, 6 * bt), :] = pool1(h1[2], h1[3])


@jax.jit
def kernel(x, w1, b1, w2, b2, f1w, f1b, f2w, f2b):
    B = x.shape[0]
    x_t = (x.reshape(B, 7, 4, 28).transpose(2, 1, 0, 3)
           .astype(jnp.bfloat16))

    w1p = jnp.pad(w1.reshape(5, 5, 32), ((0, 0), (0, 1), (0, 0)))
    ow1 = jnp.concatenate([jnp.arange(0, 24, 2), jnp.arange(1, 24, 2)])
    kw1 = jnp.arange(28)[:, None] - ow1[None, :]
    kw1 = jnp.where((kw1 >= 0) & (kw1 < 5), kw1, 5)
    t1 = w1p[:, kw1, :].reshape(140, 768).astype(jnp.bfloat16)
    b1r = jnp.tile(b1, (1, 12))

    w2p = jnp.pad(w2.reshape(5, 5, 32, 64), ((0, 0), (0, 1), (0, 0), (0, 0)))
    ow2 = jnp.array([0, 2, 1, 3])
    kw2 = jnp.arange(8)[:, None] - ow2[None, :]
    kw2 = jnp.where((kw2 >= 0) & (kw2 < 5), kw2, 5)
    t2 = (w2p[:, kw2, :, :].transpose(0, 1, 3, 2, 4)
          .reshape(1280, 256).astype(jnp.bfloat16))
    b2r = jnp.tile(b2, (1, 4))

    f1wb = f1w.astype(jnp.bfloat16)
    f2wb = f2w.astype(jnp.bfloat16)

    bt = 256
    while B % bt:
        bt //= 2
    n = B // bt
    grid = (n + 1,)

    def full(a):
        return pl.BlockSpec(a.shape, lambda i, m=a.ndim: (0,) * m)

    operands = (x_t, t1, b1r, t2, b2r, f1wb, f1b, f2wb, f2b)
    in_specs = [pl.BlockSpec((4, 7, bt, 28),
                             lambda i: (0, 0, jnp.minimum(i, n - 1), 0))]
    in_specs += [full(a) for a in operands[1:]]

    out = pl.pallas_call(
        _lenet_body,
        out_shape=jax.ShapeDtypeStruct((B, 1, 128), jnp.float32),
        grid=grid,
        in_specs=in_specs,
        out_specs=pl.BlockSpec((bt, 1, 128),
                               lambda i: (jnp.maximum(i - 1, 0), 0, 0)),
        scratch_shapes=[pltpu.VMEM((24 * bt, 384), jnp.bfloat16)],
        compiler_params=pltpu.CompilerParams(
            dimension_semantics=("arbitrary",)),
    )(*operands)

    return out[:, 0, :10]


# speedup vs baseline: 1.0151x; 1.0151x over previous
import jax
import jax.numpy as jnp
from jax.experimental import pallas as pl
from jax.experimental.pallas import tpu as pltpu


def _lenet_body(x_ref, t1_ref, b1_ref, t2_ref, b2_ref,
                f1w_ref, f1b_ref, f2w_ref, f2b_ref, o_ref, scr_ref):
    bt = x_ref.shape[2]
    cur = 0
    prev = 0

    def a1(s):
        pieces = [x_ref[(s + kh) % 4, (s + kh) // 4:(s + kh) // 4 + 6]
                  .reshape(6 * bt, 28) for kh in range(5)]
        return jnp.concatenate(pieces, axis=-1)

    t1 = t1_ref[...]
    h1 = [jnp.dot(a1(s), t1, preferred_element_type=jnp.float32)
          for s in range(4)]

    def pool1(ha, hb):
        m = jnp.maximum(ha, hb)
        m = jnp.maximum(m[:, :384], m[:, 384:])
        return jnp.maximum(m + b1_ref[...], 0.0).astype(jnp.bfloat16)

    scr_ref[pl.ds(cur, 6 * bt), :] = pool1(h1[0], h1[1])
    scr_ref[pl.ds(cur + 6 * bt, 6 * bt), :] = pool1(h1[2], h1[3])

    def a2(u, owg):
        pieces = [scr_ref[pl.ds(prev + ((u + kh) % 2) * 6 * bt
                                + ((u + kh) // 2) * bt, 4 * bt),
                          128 * owg:128 * owg + 256]
                  for kh in range(5)]
        return jnp.concatenate(pieces, axis=-1)

    t2 = t2_ref[...]
    h2 = [[jnp.dot(a2(u, owg), t2, preferred_element_type=jnp.float32)
           for owg in range(2)] for u in range(2)]
    q2 = []
    for owg in range(2):
        m2 = jnp.maximum(h2[0][owg], h2[1][owg])
        q2.append(jnp.maximum(m2[:, :128], m2[:, 128:]))
    q2 = jnp.concatenate(q2, axis=-1)
    q2 = jnp.maximum(q2 + b2_ref[...], 0.0).astype(jnp.bfloat16)

    hid = sum(jnp.dot(q2[h * bt:(h + 1) * bt, :],
                      f1w_ref[h * 256:(h + 1) * 256, :],
                      preferred_element_type=jnp.float32)
              for h in range(4))
    hid = jnp.maximum(hid + f1b_ref[...], 0.0).astype(jnp.bfloat16)
    out = jnp.dot(hid, f2w_ref[...], preferred_element_type=jnp.float32)
    o_ref[...] = (out + f2b_ref[...]).reshape(bt, 1, 128)

@jax.jit
def kernel(x, w1, b1, w2, b2, f1w, f1b, f2w, f2b):
    B = x.shape[0]
    x_t = (x.reshape(B, 7, 4, 28).transpose(2, 1, 0, 3)
           .astype(jnp.bfloat16))

    w1p = jnp.pad(w1.reshape(5, 5, 32), ((0, 0), (0, 1), (0, 0)))
    ow1 = jnp.concatenate([jnp.arange(0, 24, 2), jnp.arange(1, 24, 2)])
    kw1 = jnp.arange(28)[:, None] - ow1[None, :]
    kw1 = jnp.where((kw1 >= 0) & (kw1 < 5), kw1, 5)
    t1 = w1p[:, kw1, :].reshape(140, 768).astype(jnp.bfloat16)
    b1r = jnp.tile(b1, (1, 12))

    w2p = jnp.pad(w2.reshape(5, 5, 32, 64), ((0, 0), (0, 1), (0, 0), (0, 0)))
    ow2 = jnp.array([0, 2, 1, 3])
    kw2 = jnp.arange(8)[:, None] - ow2[None, :]
    kw2 = jnp.where((kw2 >= 0) & (kw2 < 5), kw2, 5)
    t2 = (w2p[:, kw2, :, :].transpose(0, 1, 3, 2, 4)
          .reshape(1280, 256).astype(jnp.bfloat16))
    b2r = jnp.tile(b2, (1, 4))

    f1wb = f1w.astype(jnp.bfloat16)
    f2wb = f2w.astype(jnp.bfloat16)

    bt = 256
    while B % bt:
        bt //= 2
    n = B // bt
    grid = (n,)

    def full(a):
        return pl.BlockSpec(a.shape, lambda i, m=a.ndim: (0,) * m)

    operands = (x_t, t1, b1r, t2, b2r, f1wb, f1b, f2wb, f2b)
    in_specs = [pl.BlockSpec((4, 7, bt, 28), lambda i: (0, 0, i, 0))]
    in_specs += [full(a) for a in operands[1:]]

    out = pl.pallas_call(
        _lenet_body,
        out_shape=jax.ShapeDtypeStruct((B, 1, 128), jnp.float32),
        grid=grid,
        in_specs=in_specs,
        out_specs=pl.BlockSpec((bt, 1, 128), lambda i: (i, 0, 0)),
        scratch_shapes=[pltpu.VMEM((12 * bt, 384), jnp.bfloat16)],
        compiler_params=pltpu.CompilerParams(
            dimension_semantics=("arbitrary",)),
    )(*operands)

    return out[:, 0, :10]
